# Initial kernel scaffold; baseline (speedup 1.0000x reference)
import jax
import jax.numpy as jnp
from jax.experimental import pallas as pl


def kernel(idx, emb1, emb2, W1, b1, W2, b2, alpha):
    raise NotImplementedError("write your pallas kernel here")


# speedup vs baseline: 15.2561x; 15.2561x over previous
import jax
import jax.numpy as jnp
from jax.experimental import pallas as pl
from jax.experimental.pallas import tpu as pltpu

_K = 32
_ONE_BITS_PLUS = 0x3F800001


def _nodevec_body(alpha_ref, e1_ref, w1_ref, b1_ref, e2_ref, w2_ref, b2_ref,
                  nv1_ref, nv2_ref):
    alpha = alpha_ref[0, 0]
    x1 = jnp.dot(e1_ref[...], w1_ref[...], preferred_element_type=jnp.float32)
    nv1_ref[...] = jnp.tanh(alpha * (x1 + b1_ref[...]))
    x2 = jnp.dot(e2_ref[...], w2_ref[...], preferred_element_type=jnp.float32)
    nv2_ref[...] = jnp.tanh(alpha * (x2 + b2_ref[...]))


def _cumsum_excl_lanes(x, width):
    orig = x
    shift = 1
    while shift < width:
        shifted = jnp.concatenate(
            [jnp.zeros((x.shape[0], shift), x.dtype), x[:, :width - shift]],
            axis=1)
        x = x + shifted
        shift *= 2
    return x - orig


def _adj_body(alpha_ref, n1b_ref, n2b_ref, nv1_ref, nv2_ref, out_ref,
              adj_ref, t_ref, g_ref):
    alpha = alpha_ref[0, 0]
    R = n1b_ref.shape[0]
    N = nv1_ref.shape[0]
    dn = (((1,), (1,)), ((), ()))
    d1 = jax.lax.dot_general(n1b_ref[...], nv2_ref[...], dn,
                             preferred_element_type=jnp.float32)
    d2 = jax.lax.dot_general(n2b_ref[...], nv1_ref[...], dn,
                             preferred_element_type=jnp.float32)
    adj = jnp.maximum(jnp.tanh(alpha * (d1 - d2)), 0.0)
    adj_ref[...] = adj

    c1 = jnp.sum((adj == 1.0).astype(jnp.float32), axis=1, keepdims=True)
    all_sat = jnp.all(c1 >= float(_K))

    @pl.when(all_sat)
    def _fast():
        t_ref[...] = jnp.ones((R, 1), jnp.float32)
        g_ref[...] = jnp.zeros((R, 1), jnp.float32)

    @pl.when(jnp.logical_not(all_sat))
    def _general():
        av = adj_ref[...]

        def body(_, carry):
            lo, hi = carry
            mid = jax.lax.shift_right_logical(lo + hi, 1)
            tv = jax.lax.bitcast_convert_type(mid, jnp.float32)
            cnt = jnp.sum((av >= tv).astype(jnp.float32), axis=1,
                          keepdims=True)
            ge = cnt >= float(_K)
            return (jnp.where(ge, mid, lo), jnp.where(ge, hi, mid))

        lo0 = jnp.zeros((R, 1), jnp.int32)
        hi0 = jnp.full((R, 1), _ONE_BITS_PLUS, jnp.int32)
        lo, _ = jax.lax.fori_loop(0, 31, body, (lo0, hi0))
        tv = jax.lax.bitcast_convert_type(lo, jnp.float32)
        t_ref[...] = tv
        g_ref[...] = jnp.sum((av > tv).astype(jnp.float32), axis=1,
                             keepdims=True)

    adj2 = adj_ref[...]
    tv = t_ref[...]
    gv = g_ref[...]
    eq = (adj2 == tv).astype(jnp.float32)
    prefix = _cumsum_excl_lanes(eq, N)
    keep = (adj2 > tv) | ((adj2 == tv) & (gv + prefix < float(_K)))
    out_ref[...] = jnp.where(keep, adj2, 0.0)


def kernel(idx, emb1, emb2, W1, b1, W2, b2, alpha):
    e1 = jnp.take(emb1, idx, axis=0)
    e2 = jnp.take(emb2, idx, axis=0)
    N, dim = e1.shape
    alpha2d = jnp.reshape(alpha.astype(jnp.float32), (1, 1))

    full = lambda s: pl.BlockSpec(s, lambda *_: tuple(0 for _ in s))
    smem_spec = pl.BlockSpec(memory_space=pltpu.SMEM)

    nv1, nv2 = pl.pallas_call(
        _nodevec_body,
        out_shape=[jax.ShapeDtypeStruct((N, dim), jnp.float32)] * 2,
        in_specs=[smem_spec, full((N, dim)), full((dim, dim)),
                  full((1, dim)), full((N, dim)), full((dim, dim)),
                  full((1, dim))],
        out_specs=[full((N, dim))] * 2,
    )(alpha2d, e1, W1.T, b1.reshape(1, dim), e2, W2.T, b2.reshape(1, dim))

    R = 40 if N % 40 == 0 else (8 if N % 8 == 0 else N)
    nb = N // R

    out = pl.pallas_call(
        _adj_body,
        grid=(nb,),
        out_shape=jax.ShapeDtypeStruct((N, N), jnp.float32),
        in_specs=[smem_spec,
                  pl.BlockSpec((R, dim), lambda i: (i, 0)),
                  pl.BlockSpec((R, dim), lambda i: (i, 0)),
                  full((N, dim)), full((N, dim))],
        out_specs=pl.BlockSpec((R, N), lambda i: (i, 0)),
        scratch_shapes=[pltpu.VMEM((R, N), jnp.float32),
                        pltpu.VMEM((R, 1), jnp.float32),
                        pltpu.VMEM((R, 1), jnp.float32)],
    )(alpha2d, nv1, nv2, nv1, nv2)
    return out


# speedup vs baseline: 19.2707x; 1.2631x over previous
import jax
import jax.numpy as jnp
from jax.experimental import pallas as pl
from jax.experimental.pallas import tpu as pltpu

_K = 32
_ONE_BITS_PLUS = 0x3F800001


def _nodevec_body(alpha_ref, e1_ref, w1_ref, b1_ref, e2_ref, w2_ref, b2_ref,
                  nv1_ref, nv2_ref):
    alpha = alpha_ref[0, 0]
    x1 = jnp.dot(e1_ref[...], w1_ref[...], preferred_element_type=jnp.float32)
    nv1_ref[...] = jnp.tanh(alpha * (x1 + b1_ref[...]))
    x2 = jnp.dot(e2_ref[...], w2_ref[...], preferred_element_type=jnp.float32)
    nv2_ref[...] = jnp.tanh(alpha * (x2 + b2_ref[...]))


def _cumsum_excl_lanes(x, width):
    orig = x
    shift = 1
    while shift < width:
        shifted = jnp.concatenate(
            [jnp.zeros((x.shape[0], shift), x.dtype), x[:, :width - shift]],
            axis=1)
        x = x + shifted
        shift *= 2
    return x - orig


_CW = 2048


def _col_chunks(n):
    return [(c, min(_CW, n - c)) for c in range(0, n, _CW)]


def _adj_body(alpha_ref, n1b_ref, n2b_ref, nv1_ref, nv2_ref, out_ref,
              adj_ref, t_ref, g_ref):
    alpha = alpha_ref[0, 0]
    R = n1b_ref.shape[0]
    N = nv1_ref.shape[0]
    chunks = _col_chunks(N)
    dn = (((1,), (1,)), ((), ()))

    c1 = jnp.zeros((R, 1), jnp.float32)
    for c0, cw in chunks:
        d1 = jax.lax.dot_general(n1b_ref[...], nv2_ref[c0:c0 + cw, :], dn,
                                 preferred_element_type=jnp.float32)
        d2 = jax.lax.dot_general(n2b_ref[...], nv1_ref[c0:c0 + cw, :], dn,
                                 preferred_element_type=jnp.float32)
        blk = jnp.maximum(jnp.tanh(alpha * (d1 - d2)), 0.0)
        adj_ref[:, c0:c0 + cw] = blk
        c1 = c1 + jnp.sum((blk == 1.0).astype(jnp.float32), axis=1,
                          keepdims=True)

    all_sat = jnp.all(c1 >= float(_K))

    @pl.when(all_sat)
    def _fast():
        t_ref[...] = jnp.ones((R, 1), jnp.float32)
        g_ref[...] = jnp.zeros((R, 1), jnp.float32)

    @pl.when(jnp.logical_not(all_sat))
    def _general():
        def count_ge(tv):
            cnt = jnp.zeros((R, 1), jnp.float32)
            for c0, cw in chunks:
                cnt = cnt + jnp.sum(
                    (adj_ref[:, c0:c0 + cw] >= tv).astype(jnp.float32),
                    axis=1, keepdims=True)
            return cnt

        def body(_, carry):
            lo, hi = carry
            mid = jax.lax.shift_right_logical(lo + hi, 1)
            tv = jax.lax.bitcast_convert_type(mid, jnp.float32)
            ge = count_ge(tv) >= float(_K)
            return (jnp.where(ge, mid, lo), jnp.where(ge, hi, mid))

        lo0 = jnp.zeros((R, 1), jnp.int32)
        hi0 = jnp.full((R, 1), _ONE_BITS_PLUS, jnp.int32)
        lo, _ = jax.lax.fori_loop(0, 31, body, (lo0, hi0))
        tv = jax.lax.bitcast_convert_type(lo, jnp.float32)
        t_ref[...] = tv
        gcnt = jnp.zeros((R, 1), jnp.float32)
        for c0, cw in chunks:
            gcnt = gcnt + jnp.sum(
                (adj_ref[:, c0:c0 + cw] > tv).astype(jnp.float32),
                axis=1, keepdims=True)
        g_ref[...] = gcnt

    tv = t_ref[...]
    carry = g_ref[...]
    for c0, cw in chunks:
        blk = adj_ref[:, c0:c0 + cw]
        eqb = blk == tv
        eqf = eqb.astype(jnp.float32)
        prefix = _cumsum_excl_lanes(eqf, cw)
        keep = (blk > tv) | (eqb & (carry + prefix < float(_K)))
        out_ref[:, c0:c0 + cw] = jnp.where(keep, blk, 0.0)
        carry = carry + jnp.sum(eqf, axis=1, keepdims=True)


def kernel(idx, emb1, emb2, W1, b1, W2, b2, alpha):
    e1 = jnp.take(emb1, idx, axis=0)
    e2 = jnp.take(emb2, idx, axis=0)
    N, dim = e1.shape
    alpha2d = jnp.reshape(alpha.astype(jnp.float32), (1, 1))

    full = lambda s: pl.BlockSpec(s, lambda *_: tuple(0 for _ in s))
    smem_spec = pl.BlockSpec(memory_space=pltpu.SMEM)

    nv1, nv2 = pl.pallas_call(
        _nodevec_body,
        out_shape=[jax.ShapeDtypeStruct((N, dim), jnp.float32)] * 2,
        in_specs=[smem_spec, full((N, dim)), full((dim, dim)),
                  full((1, dim)), full((N, dim)), full((dim, dim)),
                  full((1, dim))],
        out_specs=[full((N, dim))] * 2,
    )(alpha2d, e1, W1.T, b1.reshape(1, dim), e2, W2.T, b2.reshape(1, dim))

    R = 200 if N % 200 == 0 else (8 if N % 8 == 0 else N)
    nb = N // R

    out = pl.pallas_call(
        _adj_body,
        grid=(nb,),
        out_shape=jax.ShapeDtypeStruct((N, N), jnp.float32),
        in_specs=[smem_spec,
                  pl.BlockSpec((R, dim), lambda i: (i, 0)),
                  pl.BlockSpec((R, dim), lambda i: (i, 0)),
                  full((N, dim)), full((N, dim))],
        out_specs=pl.BlockSpec((R, N), lambda i: (i, 0)),
        scratch_shapes=[pltpu.VMEM((R, N), jnp.float32),
                        pltpu.VMEM((R, 1), jnp.float32),
                        pltpu.VMEM((R, 1), jnp.float32)],
    )(alpha2d, nv1, nv2, nv1, nv2)
    return out


# speedup vs baseline: 32.5314x; 1.6881x over previous
import jax
import jax.numpy as jnp
from jax.experimental import pallas as pl
from jax.experimental.pallas import tpu as pltpu

_K = 32
_ONE_BITS_PLUS = 0x3F800001
_CW = 1024


def _nodevec_body(alpha_ref, e1_ref, w1_ref, b1_ref, e2_ref, w2_ref, b2_ref,
                  nv1_ref, nv2_ref):
    alpha = alpha_ref[0, 0]
    x1 = jnp.dot(e1_ref[...], w1_ref[...], preferred_element_type=jnp.float32)
    nv1_ref[...] = jnp.tanh(alpha * (x1 + b1_ref[...]))
    x2 = jnp.dot(e2_ref[...], w2_ref[...], preferred_element_type=jnp.float32)
    nv2_ref[...] = jnp.tanh(alpha * (x2 + b2_ref[...]))


def _cumsum_excl_lanes(x, width):
    orig = x
    shift = 1
    while shift < width:
        shifted = jnp.concatenate(
            [jnp.zeros((x.shape[0], shift), x.dtype), x[:, :width - shift]],
            axis=1)
        x = x + shifted
        shift *= 2
    return x - orig


def _col_chunks(n):
    return [(c, min(_CW, n - c)) for c in range(0, n, _CW)]


def _adj_body(alpha_ref, n1b_ref, n2b_ref, nv1_ref, nv2_ref, out_ref,
              adj_ref, t_ref, g_ref, c_ref, xs_ref):
    alpha = alpha_ref[0, 0]
    R = n1b_ref.shape[0]
    N = nv1_ref.shape[0]
    chunks = _col_chunks(N)
    dn = (((1,), (1,)), ((), ()))

    @pl.when(pl.program_id(0) == 0)
    def _find_xsat():
        def bis(_, carry):
            lo, hi = carry
            mid = jax.lax.shift_right_logical(lo + hi, 1)
            xv = jax.lax.bitcast_convert_type(mid, jnp.float32)
            sat = jnp.max(jnp.tanh(jnp.full((8, 128), xv))) == 1.0
            return (jnp.where(sat, lo, mid), jnp.where(sat, mid, hi))

        lo0 = jnp.int32(0x3F800000)
        hi0 = jnp.int32(0x43800000)
        _, hi = jax.lax.fori_loop(0, 27, bis, (lo0, hi0))
        xs_ref[0, 0] = jax.lax.bitcast_convert_type(hi, jnp.float32)

    xsat = xs_ref[0, 0]

    def arg_chunk(c0, cw):
        d1 = jax.lax.dot_general(n1b_ref[...], nv2_ref[c0:c0 + cw, :], dn,
                                 preferred_element_type=jnp.float32)
        d2 = jax.lax.dot_general(n2b_ref[...], nv1_ref[c0:c0 + cw, :], dn,
                                 preferred_element_type=jnp.float32)
        return alpha * (d1 - d2)

    c1 = jnp.zeros((R, 1), jnp.float32)
    for c0, cw in chunks:
        sat = (arg_chunk(c0, cw) >= xsat).astype(jnp.float32)
        c1 = c1 + jnp.sum(sat, axis=1, keepdims=True)
    all_sat = jnp.all(c1 >= float(_K))

    @pl.when(all_sat)
    def _fast():
        c_ref[...] = jnp.zeros((R, 1), jnp.float32)
        for c0, cw in chunks:
            done = jnp.all(c_ref[...] >= float(_K))

            @pl.when(done)
            def _zeros():
                out_ref[:, c0:c0 + cw] = jnp.zeros((R, cw), jnp.float32)

            @pl.when(jnp.logical_not(done))
            def _select():
                satb = arg_chunk(c0, cw) >= xsat
                satf = satb.astype(jnp.float32)
                prefix = _cumsum_excl_lanes(satf, cw)
                carry = c_ref[...]
                keep = satb & (carry + prefix < float(_K))
                out_ref[:, c0:c0 + cw] = jnp.where(keep, 1.0, 0.0)
                c_ref[...] = carry + jnp.sum(satf, axis=1, keepdims=True)

    @pl.when(jnp.logical_not(all_sat))
    def _general():
        for c0, cw in chunks:
            adj_ref[:, c0:c0 + cw] = jnp.maximum(
                jnp.tanh(arg_chunk(c0, cw)), 0.0)

        def count_ge(tv):
            cnt = jnp.zeros((R, 1), jnp.float32)
            for c0, cw in chunks:
                cnt = cnt + jnp.sum(
                    (adj_ref[:, c0:c0 + cw] >= tv).astype(jnp.float32),
                    axis=1, keepdims=True)
            return cnt

        def bis(_, carry):
            lo, hi = carry
            mid = jax.lax.shift_right_logical(lo + hi, 1)
            tv = jax.lax.bitcast_convert_type(mid, jnp.float32)
            ge = count_ge(tv) >= float(_K)
            return (jnp.where(ge, mid, lo), jnp.where(ge, hi, mid))

        lo0 = jnp.zeros((R, 1), jnp.int32)
        hi0 = jnp.full((R, 1), _ONE_BITS_PLUS, jnp.int32)
        lo, _ = jax.lax.fori_loop(0, 31, bis, (lo0, hi0))
        tv = jax.lax.bitcast_convert_type(lo, jnp.float32)
        t_ref[...] = tv
        gcnt = jnp.zeros((R, 1), jnp.float32)
        for c0, cw in chunks:
            gcnt = gcnt + jnp.sum(
                (adj_ref[:, c0:c0 + cw] > tv).astype(jnp.float32),
                axis=1, keepdims=True)
        g_ref[...] = gcnt

        carry = g_ref[...]
        for c0, cw in chunks:
            blk = adj_ref[:, c0:c0 + cw]
            eqb = blk == tv
            eqf = eqb.astype(jnp.float32)
            prefix = _cumsum_excl_lanes(eqf, cw)
            keep = (blk > tv) | (eqb & (carry + prefix < float(_K)))
            out_ref[:, c0:c0 + cw] = jnp.where(keep, blk, 0.0)
            carry = carry + jnp.sum(eqf, axis=1, keepdims=True)


def kernel(idx, emb1, emb2, W1, b1, W2, b2, alpha):
    e1 = jnp.take(emb1, idx, axis=0)
    e2 = jnp.take(emb2, idx, axis=0)
    N, dim = e1.shape
    alpha2d = jnp.reshape(alpha.astype(jnp.float32), (1, 1))

    full = lambda s: pl.BlockSpec(s, lambda *_: tuple(0 for _ in s))
    smem_spec = pl.BlockSpec(memory_space=pltpu.SMEM)

    nv1, nv2 = pl.pallas_call(
        _nodevec_body,
        out_shape=[jax.ShapeDtypeStruct((N, dim), jnp.float32)] * 2,
        in_specs=[smem_spec, full((N, dim)), full((dim, dim)),
                  full((1, dim)), full((N, dim)), full((dim, dim)),
                  full((1, dim))],
        out_specs=[full((N, dim))] * 2,
    )(alpha2d, e1, W1.T, b1.reshape(1, dim), e2, W2.T, b2.reshape(1, dim))

    R = 80 if N % 80 == 0 else (8 if N % 8 == 0 else N)
    nb = N // R

    out = pl.pallas_call(
        _adj_body,
        grid=(nb,),
        out_shape=jax.ShapeDtypeStruct((N, N), jnp.float32),
        in_specs=[smem_spec,
                  pl.BlockSpec((R, dim), lambda i: (i, 0)),
                  pl.BlockSpec((R, dim), lambda i: (i, 0)),
                  full((N, dim)), full((N, dim))],
        out_specs=pl.BlockSpec((R, N), lambda i: (i, 0)),
        scratch_shapes=[pltpu.VMEM((R, N), jnp.float32),
                        pltpu.VMEM((R, 1), jnp.float32),
                        pltpu.VMEM((R, 1), jnp.float32),
                        pltpu.VMEM((R, 1), jnp.float32),
                        pltpu.SMEM((1, 1), jnp.float32)],
    )(alpha2d, nv1, nv2, nv1, nv2)
    return out


# speedup vs baseline: 37.9856x; 1.1677x over previous
import jax
import jax.numpy as jnp
from jax.experimental import pallas as pl
from jax.experimental.pallas import tpu as pltpu

_K = 32
_ONE_BITS_PLUS = 0x3F800001
_CW = 1024


def _nodevec_body(alpha_ref, e1_ref, w1_ref, b1_ref, e2_ref, w2_ref, b2_ref,
                  nv1_ref, nv2_ref):
    alpha = alpha_ref[0, 0]
    x1 = jnp.dot(e1_ref[...], w1_ref[...], preferred_element_type=jnp.float32)
    nv1_ref[...] = jnp.tanh(alpha * (x1 + b1_ref[...]))
    x2 = jnp.dot(e2_ref[...], w2_ref[...], preferred_element_type=jnp.float32)
    nv2_ref[...] = jnp.tanh(alpha * (x2 + b2_ref[...]))


def _cumsum_excl_lanes(x, width):
    orig = x
    shift = 1
    while shift < width:
        shifted = jnp.concatenate(
            [jnp.zeros((x.shape[0], shift), x.dtype), x[:, :width - shift]],
            axis=1)
        x = x + shifted
        shift *= 2
    return x - orig


def _col_chunks(n):
    return [(c, min(_CW, n - c)) for c in range(0, n, _CW)]


def _adj_body(alpha_ref, n1b_ref, n2b_ref, nv1_ref, nv2_ref, out_ref,
              adj_ref, t_ref, g_ref, c_ref, c1_ref, s0_ref, xs_ref):
    alpha = alpha_ref[0, 0]
    R = n1b_ref.shape[0]
    N = nv1_ref.shape[0]
    chunks = _col_chunks(N)
    dn = (((1,), (1,)), ((), ()))

    @pl.when(pl.program_id(0) == 0)
    def _find_xsat():
        def bis(_, carry):
            lo, hi = carry
            mid = jax.lax.shift_right_logical(lo + hi, 1)
            xv = jax.lax.bitcast_convert_type(mid, jnp.float32)
            sat = jnp.max(jnp.tanh(jnp.full((8, 128), xv))) == 1.0
            return (jnp.where(sat, lo, mid), jnp.where(sat, mid, hi))

        lo0 = jnp.int32(0x3F800000)
        hi0 = jnp.int32(0x43800000)
        _, hi = jax.lax.fori_loop(0, 27, bis, (lo0, hi0))
        xs_ref[0, 0] = jax.lax.bitcast_convert_type(hi, jnp.float32)

    xsat = xs_ref[0, 0]

    def arg_chunk(c0, cw):
        d1 = jax.lax.dot_general(n1b_ref[...], nv2_ref[c0:c0 + cw, :], dn,
                                 preferred_element_type=jnp.float32)
        d2 = jax.lax.dot_general(n2b_ref[...], nv1_ref[c0:c0 + cw, :], dn,
                                 preferred_element_type=jnp.float32)
        return alpha * (d1 - d2)

    for ci, (c0, cw) in enumerate(chunks):
        if ci == 0:
            satf = (arg_chunk(c0, cw) >= xsat).astype(jnp.float32)
            s0_ref[:, 0:cw] = satf
            c1_ref[...] = jnp.sum(satf, axis=1, keepdims=True)
        else:
            @pl.when(jnp.logical_not(jnp.all(c1_ref[...] >= float(_K))))
            def _count_more():
                satf = (arg_chunk(c0, cw) >= xsat).astype(jnp.float32)
                c1_ref[...] = c1_ref[...] + jnp.sum(satf, axis=1,
                                                    keepdims=True)
    all_sat = jnp.all(c1_ref[...] >= float(_K))

    @pl.when(all_sat)
    def _fast():
        c_ref[...] = jnp.zeros((R, 1), jnp.float32)
        for ci, (c0, cw) in enumerate(chunks):
            done = jnp.all(c_ref[...] >= float(_K))

            @pl.when(done)
            def _zeros():
                out_ref[:, c0:c0 + cw] = jnp.zeros((R, cw), jnp.float32)

            @pl.when(jnp.logical_not(done))
            def _select(ci=ci):
                satf = (s0_ref[:, 0:cw] if ci == 0 else
                        (arg_chunk(c0, cw) >= xsat).astype(jnp.float32))
                prefix = _cumsum_excl_lanes(satf, cw)
                carry = c_ref[...]
                keep = (satf > 0.0) & (carry + prefix < float(_K))
                out_ref[:, c0:c0 + cw] = jnp.where(keep, 1.0, 0.0)
                c_ref[...] = carry + jnp.sum(satf, axis=1, keepdims=True)

    @pl.when(jnp.logical_not(all_sat))
    def _general():
        for c0, cw in chunks:
            adj_ref[:, c0:c0 + cw] = jnp.maximum(
                jnp.tanh(arg_chunk(c0, cw)), 0.0)

        def count_ge(tv):
            cnt = jnp.zeros((R, 1), jnp.float32)
            for c0, cw in chunks:
                cnt = cnt + jnp.sum(
                    (adj_ref[:, c0:c0 + cw] >= tv).astype(jnp.float32),
                    axis=1, keepdims=True)
            return cnt

        def bis(_, carry):
            lo, hi = carry
            mid = jax.lax.shift_right_logical(lo + hi, 1)
            tv = jax.lax.bitcast_convert_type(mid, jnp.float32)
            ge = count_ge(tv) >= float(_K)
            return (jnp.where(ge, mid, lo), jnp.where(ge, hi, mid))

        lo0 = jnp.zeros((R, 1), jnp.int32)
        hi0 = jnp.full((R, 1), _ONE_BITS_PLUS, jnp.int32)
        lo, _ = jax.lax.fori_loop(0, 31, bis, (lo0, hi0))
        tv = jax.lax.bitcast_convert_type(lo, jnp.float32)
        t_ref[...] = tv
        gcnt = jnp.zeros((R, 1), jnp.float32)
        for c0, cw in chunks:
            gcnt = gcnt + jnp.sum(
                (adj_ref[:, c0:c0 + cw] > tv).astype(jnp.float32),
                axis=1, keepdims=True)
        g_ref[...] = gcnt

        carry = g_ref[...]
        for c0, cw in chunks:
            blk = adj_ref[:, c0:c0 + cw]
            eqb = blk == tv
            eqf = eqb.astype(jnp.float32)
            prefix = _cumsum_excl_lanes(eqf, cw)
            keep = (blk > tv) | (eqb & (carry + prefix < float(_K)))
            out_ref[:, c0:c0 + cw] = jnp.where(keep, blk, 0.0)
            carry = carry + jnp.sum(eqf, axis=1, keepdims=True)


def kernel(idx, emb1, emb2, W1, b1, W2, b2, alpha):
    e1 = jnp.take(emb1, idx, axis=0)
    e2 = jnp.take(emb2, idx, axis=0)
    N, dim = e1.shape
    alpha2d = jnp.reshape(alpha.astype(jnp.float32), (1, 1))

    full = lambda s: pl.BlockSpec(s, lambda *_: tuple(0 for _ in s))
    smem_spec = pl.BlockSpec(memory_space=pltpu.SMEM)

    nv1, nv2 = pl.pallas_call(
        _nodevec_body,
        out_shape=[jax.ShapeDtypeStruct((N, dim), jnp.float32)] * 2,
        in_specs=[smem_spec, full((N, dim)), full((dim, dim)),
                  full((1, dim)), full((N, dim)), full((dim, dim)),
                  full((1, dim))],
        out_specs=[full((N, dim))] * 2,
    )(alpha2d, e1, W1.T, b1.reshape(1, dim), e2, W2.T, b2.reshape(1, dim))

    R = 80 if N % 80 == 0 else (8 if N % 8 == 0 else N)
    nb = N // R

    out = pl.pallas_call(
        _adj_body,
        grid=(nb,),
        out_shape=jax.ShapeDtypeStruct((N, N), jnp.float32),
        in_specs=[smem_spec,
                  pl.BlockSpec((R, dim), lambda i: (i, 0)),
                  pl.BlockSpec((R, dim), lambda i: (i, 0)),
                  full((N, dim)), full((N, dim))],
        out_specs=pl.BlockSpec((R, N), lambda i: (i, 0)),
        scratch_shapes=[pltpu.VMEM((R, N), jnp.float32),
                        pltpu.VMEM((R, 1), jnp.float32),
                        pltpu.VMEM((R, 1), jnp.float32),
                        pltpu.VMEM((R, 1), jnp.float32),
                        pltpu.VMEM((R, 1), jnp.float32),
                        pltpu.VMEM((R, _CW), jnp.float32),
                        pltpu.SMEM((1, 1), jnp.float32)],
    )(alpha2d, nv1, nv2, nv1, nv2)
    return out
